# baseline (speedup 1.0000x reference)
import functools

import jax
import jax.numpy as jnp
from jax import lax
from jax.experimental import pallas as pl
from jax.experimental.pallas import tpu as pltpu
from jax.experimental.pallas import tpu_sc as plsc

NC = 2
NS = 16
NW = NC * NS
BATCH = 16384
VOCAB = 1000000
EMBED = 64
B_PER_W = BATCH // NW
CHUNK = 128
NCHUNK = B_PER_W // CHUNK
BLK = 65536
NBLK = 16
VP = NBLK * BLK


def _stage_a_body(tt_ref, w_ref, bh_ref, o0_ref, o1_ref, o2_ref, o3_ref):
    x = tt_ref[...]
    ss = jnp.sum(x * x, axis=0, keepdims=True)
    s = lax.rsqrt(jnp.maximum(ss, 1.0))
    p = lax.dot_general(w_ref[...], x, (((1,), (0,)), ((), ())),
                        preferred_element_type=jnp.float32)
    ps = p * s + bh_ref[...]
    o0_ref[...] = ps[0]
    o1_ref[...] = ps[1]
    o2_ref[...] = ps[2]
    o3_ref[...] = ps[3]


def _stage_a(tt, w4, bh):
    grid = (NBLK,)
    plane = jax.ShapeDtypeStruct((VP,), jnp.float32)
    return pl.pallas_call(
        _stage_a_body,
        grid=grid,
        in_specs=[
            pl.BlockSpec((EMBED, BLK), lambda i: (0, i)),
            pl.BlockSpec((8, EMBED), lambda i: (0, 0)),
            pl.BlockSpec((8, 1), lambda i: (0, 0)),
        ],
        out_specs=[pl.BlockSpec((BLK,), lambda i: (i,))] * 4,
        out_shape=[plane] * 4,
        compiler_params=pltpu.CompilerParams(
            dimension_semantics=("parallel",)),
    )(tt, w4, bh)


def _stage_b(p0, p1, p2, p3, head, tail):
    mesh = plsc.VectorSubcoreMesh(core_axis_name="c", subcore_axis_name="s")

    @functools.partial(
        pl.kernel,
        out_type=jax.ShapeDtypeStruct((2, BATCH), jnp.float32),
        mesh=mesh,
        scratch_types=[
            pltpu.VMEM((B_PER_W,), jnp.int32),
            pltpu.VMEM((B_PER_W,), jnp.int32),
            pltpu.VMEM((B_PER_W,), jnp.float32),
            pltpu.VMEM((B_PER_W,), jnp.float32),
            pltpu.VMEM((B_PER_W,), jnp.float32),
            pltpu.VMEM((B_PER_W,), jnp.float32),
            pltpu.VMEM((B_PER_W,), jnp.float32),
            pltpu.VMEM((B_PER_W,), jnp.float32),
            pltpu.SemaphoreType.DMA,
        ],
        compiler_params=pltpu.CompilerParams(use_tc_tiling_on_sc=False),
    )
    def k(p0_hbm, p1_hbm, p2_hbm, p3_hbm, h_hbm, t_hbm, out_hbm,
          hidx_v, tidx_v, g0_v, g1_v, g2_v, g3_v, o0_v, o1_v, sem):
        wid = lax.axis_index("s") * NC + lax.axis_index("c")
        base = wid * B_PER_W
        pltpu.sync_copy(h_hbm.at[pl.ds(base, B_PER_W)], hidx_v)
        pltpu.sync_copy(t_hbm.at[pl.ds(base, B_PER_W)], tidx_v)
        copies = []
        for c in range(NCHUNK):
            sl = pl.ds(c * CHUNK, CHUNK)
            copies.append(pltpu.async_copy(
                p0_hbm.at[hidx_v.at[sl]], g0_v.at[sl], sem))
            copies.append(pltpu.async_copy(
                p1_hbm.at[hidx_v.at[sl]], g1_v.at[sl], sem))
            copies.append(pltpu.async_copy(
                p2_hbm.at[tidx_v.at[sl]], g2_v.at[sl], sem))
            copies.append(pltpu.async_copy(
                p3_hbm.at[tidx_v.at[sl]], g3_v.at[sl], sem))
        for cp in copies:
            cp.wait()

        @pl.loop(0, B_PER_W, step=16)
        def _(i):
            sl = pl.ds(i, 16)
            o0_v[sl] = g0_v[sl] + g2_v[sl]
            o1_v[sl] = g1_v[sl] + g3_v[sl]

        pltpu.sync_copy(o0_v, out_hbm.at[0, pl.ds(base, B_PER_W)])
        pltpu.sync_copy(o1_v, out_hbm.at[1, pl.ds(base, B_PER_W)])

    return k(p0, p1, p2, p3, head, tail)


def kernel(head, tail, table, W, b):
    tt = table.T
    w4 = jnp.concatenate(
        [W[:, :EMBED], W[:, EMBED:], jnp.zeros((4, EMBED), jnp.float32)],
        axis=0)
    bh = (0.5 * jnp.concatenate([b, b, jnp.zeros((4,), jnp.float32)])
          ).reshape(8, 1)
    p0, p1, p2, p3 = _stage_a(tt, w4, bh)
    out_t = _stage_b(p0, p1, p2, p3, head, tail)
    return out_t.T


# speedup vs baseline: 1.0182x; 1.0182x over previous
import functools

import jax
import jax.numpy as jnp
from jax import lax
from jax.experimental import pallas as pl
from jax.experimental.pallas import tpu as pltpu
from jax.experimental.pallas import tpu_sc as plsc

NC = 2
NS = 16
NW = NC * NS
BATCH = 16384
VOCAB = 1000000
EMBED = 64
B_PER_W = BATCH // NW
CHUNK = 128
NCHUNK = B_PER_W // CHUNK
BLK = 32768
NBLK = 31
VP = NBLK * BLK


def _stage_a_body(tt_ref, w_ref, bh_ref, o0_ref, o1_ref, o2_ref, o3_ref):
    x = tt_ref[...]
    ss = jnp.sum(x * x, axis=0, keepdims=True)
    s = lax.rsqrt(jnp.maximum(ss, 1.0))
    p = lax.dot_general(w_ref[...], x, (((1,), (0,)), ((), ())),
                        preferred_element_type=jnp.float32)
    ps = p * s + bh_ref[...]
    o0_ref[...] = ps[0]
    o1_ref[...] = ps[1]
    o2_ref[...] = ps[2]
    o3_ref[...] = ps[3]


def _stage_a(tt, w4, bh):
    grid = (NBLK,)
    plane = jax.ShapeDtypeStruct((VP,), jnp.float32)
    return pl.pallas_call(
        _stage_a_body,
        grid=grid,
        in_specs=[
            pl.BlockSpec((EMBED, BLK), lambda i: (0, i)),
            pl.BlockSpec((8, EMBED), lambda i: (0, 0)),
            pl.BlockSpec((8, 1), lambda i: (0, 0)),
        ],
        out_specs=[pl.BlockSpec((BLK,), lambda i: (i,))] * 4,
        out_shape=[plane] * 4,
        compiler_params=pltpu.CompilerParams(
            dimension_semantics=("parallel",)),
    )(tt, w4, bh)


def _stage_b(p0, p1, p2, p3, head, tail):
    mesh = plsc.VectorSubcoreMesh(core_axis_name="c", subcore_axis_name="s")

    @functools.partial(
        pl.kernel,
        out_type=jax.ShapeDtypeStruct((2, BATCH), jnp.float32),
        mesh=mesh,
        scratch_types=[
            pltpu.VMEM((B_PER_W,), jnp.int32),
            pltpu.VMEM((B_PER_W,), jnp.int32),
            pltpu.VMEM((B_PER_W,), jnp.float32),
            pltpu.VMEM((B_PER_W,), jnp.float32),
            pltpu.VMEM((B_PER_W,), jnp.float32),
            pltpu.VMEM((B_PER_W,), jnp.float32),
            pltpu.VMEM((B_PER_W,), jnp.float32),
            pltpu.VMEM((B_PER_W,), jnp.float32),
            pltpu.SemaphoreType.DMA,
        ],
        compiler_params=pltpu.CompilerParams(use_tc_tiling_on_sc=False),
    )
    def k(p0_hbm, p1_hbm, p2_hbm, p3_hbm, h_hbm, t_hbm, out_hbm,
          hidx_v, tidx_v, g0_v, g1_v, g2_v, g3_v, o0_v, o1_v, sem):
        wid = lax.axis_index("s") * NC + lax.axis_index("c")
        base = wid * B_PER_W
        pltpu.sync_copy(h_hbm.at[pl.ds(base, B_PER_W)], hidx_v)
        pltpu.sync_copy(t_hbm.at[pl.ds(base, B_PER_W)], tidx_v)
        copies = []
        for c in range(NCHUNK):
            sl = pl.ds(c * CHUNK, CHUNK)
            copies.append(pltpu.async_copy(
                p0_hbm.at[hidx_v.at[sl]], g0_v.at[sl], sem))
            copies.append(pltpu.async_copy(
                p1_hbm.at[hidx_v.at[sl]], g1_v.at[sl], sem))
            copies.append(pltpu.async_copy(
                p2_hbm.at[tidx_v.at[sl]], g2_v.at[sl], sem))
            copies.append(pltpu.async_copy(
                p3_hbm.at[tidx_v.at[sl]], g3_v.at[sl], sem))
        for cp in copies:
            cp.wait()

        @pl.loop(0, B_PER_W, step=16)
        def _(i):
            sl = pl.ds(i, 16)
            o0_v[sl] = g0_v[sl] + g2_v[sl]
            o1_v[sl] = g1_v[sl] + g3_v[sl]

        pltpu.sync_copy(o0_v, out_hbm.at[0, pl.ds(base, B_PER_W)])
        pltpu.sync_copy(o1_v, out_hbm.at[1, pl.ds(base, B_PER_W)])

    return k(p0, p1, p2, p3, head, tail)


def kernel(head, tail, table, W, b):
    tt = table.T
    w4 = jnp.concatenate(
        [W[:, :EMBED], W[:, EMBED:], jnp.zeros((4, EMBED), jnp.float32)],
        axis=0)
    bh = (0.5 * jnp.concatenate([b, b, jnp.zeros((4,), jnp.float32)])
          ).reshape(8, 1)
    p0, p1, p2, p3 = _stage_a(tt, w4, bh)
    out_t = _stage_b(p0, p1, p2, p3, head, tail)
    return out_t.T
